# baseline (speedup 1.0000x reference)
import functools

import jax
import jax.numpy as jnp
import numpy as np
from jax import lax
from jax.experimental import pallas as pl
from jax.experimental.pallas import tpu as pltpu
from jax.experimental.pallas import tpu_sc as plsc

VOCAB = 100000
EMB = 256
HID = 2048
OUT = 4096
B = 4096
L = 200

NC = 2
NS = 16
NW = NC * NS
LANES = 16
NCH = EMB // LANES

CH0, CH1 = 128, 72


def _make_pool_body(spw):
    def _pool_body(tok_hbm, table_hbm, out_hbm, idx_v, rows_a, rows_b, accst,
                   sem_a, sem_b):
        wid = lax.axis_index("s") * NC + lax.axis_index("c")
        seg0 = wid * spw

        pltpu.sync_copy(tok_hbm.at[pl.ds(seg0 * L, spw * L)], idx_v)

        def issue(seg, rows, sem):
            off = seg * L
            pltpu.async_copy(table_hbm.at[idx_v.at[pl.ds(off, CH0)]],
                             rows.at[pl.ds(0, CH0)], sem)
            pltpu.async_copy(table_hbm.at[idx_v.at[pl.ds(off + CH0, CH1)]],
                             rows.at[pl.ds(CH0, CH1)], sem)

        def wait(seg, rows, sem):
            off = seg * L
            pltpu.make_async_copy(table_hbm.at[idx_v.at[pl.ds(off, CH0)]],
                                  rows.at[pl.ds(0, CH0)], sem).wait()
            pltpu.make_async_copy(table_hbm.at[idx_v.at[pl.ds(off + CH0, CH1)]],
                                  rows.at[pl.ds(CH0, CH1)], sem).wait()

        def acc_store(seg, rows):
            def body8(r, acc):
                acc = list(acc)
                for u in range(8):
                    for c in range(NCH // 2):
                        w = rows[r * 8 + u, pl.ds(c * LANES, LANES)]
                        a = lax.bitcast_convert_type(
                            lax.shift_left(w, 16), jnp.float32)
                        b = lax.bitcast_convert_type(
                            lax.bitwise_and(w, jnp.int32(-65536)), jnp.float32)
                        acc[2 * c] = acc[2 * c] + a
                        acc[2 * c + 1] = acc[2 * c + 1] + b
                return tuple(acc)

            acc = tuple(jnp.zeros((LANES,), jnp.float32) for _ in range(NCH))
            acc = lax.fori_loop(0, L // 8, body8, acc)
            inv = jnp.float32(1.0 / L)
            for j in range(NCH):
                accst[0, pl.ds(j * LANES, LANES)] = acc[j] * inv
            pltpu.sync_copy(accst, out_hbm.at[pl.ds(seg0 + seg, 1)])

        issue(0, rows_a, sem_a)
        issue(1, rows_b, sem_b)

        def pair(i, carry):
            sa = i * 2
            wait(sa, rows_a, sem_a)
            acc_store(sa, rows_a)
            issue(sa + 2, rows_a, sem_a)
            wait(sa + 1, rows_b, sem_b)
            acc_store(sa + 1, rows_b)
            issue(sa + 3, rows_b, sem_b)
            return carry

        lax.fori_loop(0, spw // 2 - 1, pair, 0)
        last = spw - 2
        wait(last, rows_a, sem_a)
        acc_store(last, rows_a)
        wait(last + 1, rows_b, sem_b)
        acc_store(last + 1, rows_b)

    return _pool_body


@functools.cache
def _pool(nb):
    spw = nb // NW
    return functools.partial(
        pl.kernel,
        out_type=jax.ShapeDtypeStruct((nb, EMB), jnp.float32),
        mesh=plsc.VectorSubcoreMesh(core_axis_name="c", subcore_axis_name="s",
                                    num_cores=NC, num_subcores=NS),
        scratch_types=[
            pltpu.VMEM((spw * L,), jnp.int32),
            pltpu.VMEM((L, EMB // 2), jnp.int32),
            pltpu.VMEM((L, EMB // 2), jnp.int32),
            pltpu.VMEM((1, EMB), jnp.float32),
            pltpu.SemaphoreType.DMA,
            pltpu.SemaphoreType.DMA,
        ],
    )(_make_pool_body(spw))


def _mlp_body(x_ref, w1_ref, b1_ref, g_ref, bt_ref, w2_ref, b2_ref, o_ref):
    x = x_ref[...]
    h = jnp.dot(x, w1_ref[...], preferred_element_type=jnp.float32) + b1_ref[...]
    h = 0.5 * h * (1.0 + lax.erf(h * jnp.float32(0.7071067811865476)))
    mu = jnp.mean(h, axis=-1, keepdims=True)
    hc = h - mu
    var = jnp.mean(hc * hc, axis=-1, keepdims=True)
    h = hc * lax.rsqrt(var + 1e-5)
    h = h * g_ref[...] + bt_ref[...]
    out = jnp.dot(h, w2_ref[...], preferred_element_type=jnp.float32) + b2_ref[...]
    n2 = jnp.sum(out * out, axis=-1, keepdims=True)
    o_ref[...] = out * lax.rsqrt(jnp.maximum(n2, 1e-24))


def _pack_body(t_ref, o_ref):
    x = t_ref[...]
    bits = lax.bitcast_convert_type(x, jnp.uint32)
    rnd = (bits + jnp.uint32(0x7FFF) + ((bits >> 16) & jnp.uint32(1))) >> 16
    lo = rnd[:, :EMB // 2]
    hi = rnd[:, EMB // 2:]
    o_ref[...] = lax.bitcast_convert_type(lo | (hi << 16), jnp.int32)


BLKV = 2000


def _pack(table):
    return pl.pallas_call(
        _pack_body,
        grid=(VOCAB // BLKV,),
        in_specs=[pl.BlockSpec((BLKV, EMB), lambda i: (i, 0))],
        out_specs=pl.BlockSpec((BLKV, EMB // 2), lambda i: (i, 0)),
        out_shape=jax.ShapeDtypeStruct((VOCAB, EMB // 2), jnp.int32),
    )(table)


BLK = 256


def _mlp(pooled, W1, b1, gamma, beta, W2, b2):
    nb = pooled.shape[0]
    return pl.pallas_call(
        _mlp_body,
        grid=(nb // BLK,),
        in_specs=[
            pl.BlockSpec((BLK, EMB), lambda i: (i, 0)),
            pl.BlockSpec((EMB, HID), lambda i: (0, 0)),
            pl.BlockSpec((1, HID), lambda i: (0, 0)),
            pl.BlockSpec((1, HID), lambda i: (0, 0)),
            pl.BlockSpec((1, HID), lambda i: (0, 0)),
            pl.BlockSpec((HID, OUT), lambda i: (0, 0)),
            pl.BlockSpec((1, OUT), lambda i: (0, 0)),
        ],
        out_specs=pl.BlockSpec((BLK, OUT), lambda i: (i, 0)),
        out_shape=jax.ShapeDtypeStruct((nb, OUT), jnp.float32),
    )(pooled, W1, b1, gamma, beta, W2, b2)


_PERM = np.concatenate([
    np.concatenate([np.arange(16 * c, 16 * c + 16),
                    np.arange(EMB // 2 + 16 * c, EMB // 2 + 16 * c + 16)])
    for c in range(EMB // 32)
])

NSPLIT = 2


def kernel(token_ids, table, W1, b1, gamma, beta, W2, b2):
    tok_flat = token_ids.reshape(-1).astype(jnp.int32)
    table_b = _pack(table)
    W1p = jnp.take(W1, jnp.asarray(_PERM), axis=0)
    args = (W1p, b1.reshape(1, HID), gamma.reshape(1, HID),
            beta.reshape(1, HID), W2, b2.reshape(1, OUT))
    nb = B // NSPLIT
    pooled = [_pool(nb)(tok_flat[i * nb * L:(i + 1) * nb * L], table_b)
              for i in range(NSPLIT)]
    outs = [_mlp(p, *args) for p in pooled]
    return jnp.concatenate(outs, axis=0)


# speedup vs baseline: 1.1032x; 1.1032x over previous
import functools

import jax
import jax.numpy as jnp
import numpy as np
from jax import lax
from jax.experimental import pallas as pl
from jax.experimental.pallas import tpu as pltpu
from jax.experimental.pallas import tpu_sc as plsc

VOCAB = 100000
EMB = 256
HID = 2048
OUT = 4096
B = 4096
L = 200

NC = 2
NS = 16
NW = NC * NS
LANES = 16
NCH = EMB // LANES

CH0, CH1 = 128, 72


def _make_pool_body(spw):
    def _pool_body(tok_hbm, table_hbm, out_hbm, idx_v, rows_a, rows_b, accst,
                   sem_a, sem_b):
        wid = lax.axis_index("s") * NC + lax.axis_index("c")
        seg0 = wid * spw

        pltpu.sync_copy(tok_hbm.at[pl.ds(seg0 * L, spw * L)], idx_v)

        def issue(seg, rows, sem):
            off = seg * L
            pltpu.async_copy(table_hbm.at[idx_v.at[pl.ds(off, CH0)]],
                             rows.at[pl.ds(0, CH0)], sem)
            pltpu.async_copy(table_hbm.at[idx_v.at[pl.ds(off + CH0, CH1)]],
                             rows.at[pl.ds(CH0, CH1)], sem)

        def wait(seg, rows, sem):
            off = seg * L
            pltpu.make_async_copy(table_hbm.at[idx_v.at[pl.ds(off, CH0)]],
                                  rows.at[pl.ds(0, CH0)], sem).wait()
            pltpu.make_async_copy(table_hbm.at[idx_v.at[pl.ds(off + CH0, CH1)]],
                                  rows.at[pl.ds(CH0, CH1)], sem).wait()

        def acc_store(seg, rows):
            def body8(r, acc):
                acc = list(acc)
                for u in range(8):
                    for c in range(NCH // 2):
                        w = rows[r * 8 + u, pl.ds(c * LANES, LANES)]
                        a = lax.bitcast_convert_type(
                            lax.shift_left(w, 16), jnp.float32)
                        b = lax.bitcast_convert_type(
                            lax.bitwise_and(w, jnp.int32(-65536)), jnp.float32)
                        acc[2 * c] = acc[2 * c] + a
                        acc[2 * c + 1] = acc[2 * c + 1] + b
                return tuple(acc)

            acc = tuple(jnp.zeros((LANES,), jnp.float32) for _ in range(NCH))
            acc = lax.fori_loop(0, L // 8, body8, acc)
            inv = jnp.float32(1.0 / L)
            for c in range(NCH // 2):
                accst[0, pl.ds(c * LANES, LANES)] = acc[2 * c] * inv
                accst[0, pl.ds(EMB // 2 + c * LANES, LANES)] = (
                    acc[2 * c + 1] * inv)
            pltpu.sync_copy(accst, out_hbm.at[pl.ds(seg0 + seg, 1)])

        issue(0, rows_a, sem_a)
        issue(1, rows_b, sem_b)

        def pair(i, carry):
            sa = i * 2
            wait(sa, rows_a, sem_a)
            acc_store(sa, rows_a)
            issue(sa + 2, rows_a, sem_a)
            wait(sa + 1, rows_b, sem_b)
            acc_store(sa + 1, rows_b)
            issue(sa + 3, rows_b, sem_b)
            return carry

        lax.fori_loop(0, spw // 2 - 1, pair, 0)
        last = spw - 2
        wait(last, rows_a, sem_a)
        acc_store(last, rows_a)
        wait(last + 1, rows_b, sem_b)
        acc_store(last + 1, rows_b)

    return _pool_body


@functools.cache
def _pool(nb):
    spw = nb // NW
    return functools.partial(
        pl.kernel,
        out_type=jax.ShapeDtypeStruct((nb, EMB), jnp.float32),
        mesh=plsc.VectorSubcoreMesh(core_axis_name="c", subcore_axis_name="s",
                                    num_cores=NC, num_subcores=NS),
        scratch_types=[
            pltpu.VMEM((spw * L,), jnp.int32),
            pltpu.VMEM((L, EMB // 2), jnp.int32),
            pltpu.VMEM((L, EMB // 2), jnp.int32),
            pltpu.VMEM((1, EMB), jnp.float32),
            pltpu.SemaphoreType.DMA,
            pltpu.SemaphoreType.DMA,
        ],
    )(_make_pool_body(spw))


def _mlp_body(x_ref, w1_ref, b1_ref, g_ref, bt_ref, w2_ref, b2_ref, o_ref):
    x = x_ref[...]
    h = jnp.dot(x, w1_ref[...], preferred_element_type=jnp.float32) + b1_ref[...]
    h = 0.5 * h * (1.0 + lax.erf(h * jnp.float32(0.7071067811865476)))
    mu = jnp.mean(h, axis=-1, keepdims=True)
    hc = h - mu
    var = jnp.mean(hc * hc, axis=-1, keepdims=True)
    h = hc * lax.rsqrt(var + 1e-5)
    h = h * g_ref[...] + bt_ref[...]
    out = jnp.dot(h, w2_ref[...], preferred_element_type=jnp.float32) + b2_ref[...]
    n2 = jnp.sum(out * out, axis=-1, keepdims=True)
    o_ref[...] = out * lax.rsqrt(jnp.maximum(n2, 1e-24))


def _pack_body(t_ref, o_ref):
    x = t_ref[...]
    bits = lax.bitcast_convert_type(x, jnp.uint32)
    rnd = (bits + jnp.uint32(0x7FFF) + ((bits >> 16) & jnp.uint32(1))) >> 16
    lo = rnd[:, :EMB // 2]
    hi = rnd[:, EMB // 2:]
    o_ref[...] = lax.bitcast_convert_type(lo | (hi << 16), jnp.int32)


BLKV = 2000


def _pack(table):
    return pl.pallas_call(
        _pack_body,
        grid=(VOCAB // BLKV,),
        in_specs=[pl.BlockSpec((BLKV, EMB), lambda i: (i, 0))],
        out_specs=pl.BlockSpec((BLKV, EMB // 2), lambda i: (i, 0)),
        out_shape=jax.ShapeDtypeStruct((VOCAB, EMB // 2), jnp.int32),
    )(table)


BLK = 256


def _mlp_body_prev(prev_ref, x_ref, w1_ref, b1_ref, g_ref, bt_ref, w2_ref,
                   b2_ref, o_ref):
    del prev_ref
    _mlp_body(x_ref, w1_ref, b1_ref, g_ref, bt_ref, w2_ref, b2_ref, o_ref)


_W_SPECS = [
    pl.BlockSpec((EMB, HID), lambda i: (0, 0)),
    pl.BlockSpec((1, HID), lambda i: (0, 0)),
    pl.BlockSpec((1, HID), lambda i: (0, 0)),
    pl.BlockSpec((1, HID), lambda i: (0, 0)),
    pl.BlockSpec((HID, OUT), lambda i: (0, 0)),
    pl.BlockSpec((1, OUT), lambda i: (0, 0)),
]


def _mlp_chunk(prev, pooled_q, W1, b1, gamma, beta, W2, b2, q):
    nbq = pooled_q.shape[0]
    gq = nbq // BLK
    out_spec = pl.BlockSpec((BLK, OUT), lambda i, q=q, gq=gq: (i + q * gq, 0))
    x_spec = pl.BlockSpec((BLK, EMB), lambda i: (i, 0))
    if q == 0:
        return pl.pallas_call(
            _mlp_body,
            grid=(gq,),
            in_specs=[x_spec, *_W_SPECS],
            out_specs=out_spec,
            out_shape=jax.ShapeDtypeStruct((B, OUT), jnp.float32),
        )(pooled_q, W1, b1, gamma, beta, W2, b2)
    return pl.pallas_call(
        _mlp_body_prev,
        grid=(gq,),
        in_specs=[pl.BlockSpec(memory_space=pl.ANY), x_spec, *_W_SPECS],
        out_specs=out_spec,
        out_shape=jax.ShapeDtypeStruct((B, OUT), jnp.float32),
        input_output_aliases={0: 0},
    )(prev, pooled_q, W1, b1, gamma, beta, W2, b2)


NSPLIT = 4


def kernel(token_ids, table, W1, b1, gamma, beta, W2, b2):
    tok_flat = token_ids.reshape(-1).astype(jnp.int32)
    table_b = _pack(table)
    args = (W1, b1.reshape(1, HID), gamma.reshape(1, HID),
            beta.reshape(1, HID), W2, b2.reshape(1, OUT))
    nb = B // NSPLIT
    pooled = [_pool(nb)(tok_flat[i * nb * L:(i + 1) * nb * L], table_b)
              for i in range(NSPLIT)]
    out = None
    for q in range(NSPLIT):
        out = _mlp_chunk(out, pooled[q], *args, q)
    return out


# speedup vs baseline: 1.1852x; 1.0743x over previous
import functools

import jax
import jax.numpy as jnp
import numpy as np
from jax import lax
from jax.experimental import pallas as pl
from jax.experimental.pallas import tpu as pltpu
from jax.experimental.pallas import tpu_sc as plsc

VOCAB = 100000
EMB = 256
HID = 2048
OUT = 4096
B = 4096
L = 200

NC = 2
NS = 16
NW = NC * NS
LANES = 16
NCH = EMB // LANES

CH0, CH1 = 128, 72


def _make_pool_body(spw):
    def _pool_body(tok_hbm, table_hbm, out_hbm, idx_v, rows_a, rows_b, accst,
                   sem_a, sem_b):
        wid = lax.axis_index("s") * NC + lax.axis_index("c")
        seg0 = wid * spw

        pltpu.sync_copy(tok_hbm.at[pl.ds(seg0 * L, spw * L)], idx_v)

        def issue(seg, rows, sem):
            off = seg * L
            pltpu.async_copy(table_hbm.at[idx_v.at[pl.ds(off, CH0)]],
                             rows.at[pl.ds(0, CH0)], sem)
            pltpu.async_copy(table_hbm.at[idx_v.at[pl.ds(off + CH0, CH1)]],
                             rows.at[pl.ds(CH0, CH1)], sem)

        def wait(seg, rows, sem):
            off = seg * L
            pltpu.make_async_copy(table_hbm.at[idx_v.at[pl.ds(off, CH0)]],
                                  rows.at[pl.ds(0, CH0)], sem).wait()
            pltpu.make_async_copy(table_hbm.at[idx_v.at[pl.ds(off + CH0, CH1)]],
                                  rows.at[pl.ds(CH0, CH1)], sem).wait()

        def acc_store(seg, rows):
            def body8(r, acc):
                acc = list(acc)
                for u in range(8):
                    for c in range(NCH // 2):
                        w = rows[r * 8 + u, pl.ds(c * LANES, LANES)]
                        a = lax.bitcast_convert_type(
                            lax.shift_left(w, 16), jnp.float32)
                        b = lax.bitcast_convert_type(w, jnp.float32)
                        acc[2 * c] = acc[2 * c] + a
                        acc[2 * c + 1] = acc[2 * c + 1] + b
                return tuple(acc)

            acc = tuple(jnp.zeros((LANES,), jnp.float32) for _ in range(NCH))
            acc = lax.fori_loop(0, L // 8, body8, acc)
            inv = jnp.float32(1.0 / L)
            for c in range(NCH // 2):
                accst[0, pl.ds(c * LANES, LANES)] = acc[2 * c] * inv
                accst[0, pl.ds(EMB // 2 + c * LANES, LANES)] = (
                    acc[2 * c + 1] * inv)
            pltpu.sync_copy(accst, out_hbm.at[pl.ds(seg0 + seg, 1)])

        issue(0, rows_a, sem_a)
        issue(1, rows_b, sem_b)

        def pair(i, carry):
            sa = i * 2
            wait(sa, rows_a, sem_a)
            acc_store(sa, rows_a)
            issue(sa + 2, rows_a, sem_a)
            wait(sa + 1, rows_b, sem_b)
            acc_store(sa + 1, rows_b)
            issue(sa + 3, rows_b, sem_b)
            return carry

        lax.fori_loop(0, spw // 2 - 1, pair, 0)
        last = spw - 2
        wait(last, rows_a, sem_a)
        acc_store(last, rows_a)
        wait(last + 1, rows_b, sem_b)
        acc_store(last + 1, rows_b)

    return _pool_body


@functools.cache
def _pool(nb):
    spw = nb // NW
    return functools.partial(
        pl.kernel,
        out_type=jax.ShapeDtypeStruct((nb, EMB), jnp.float32),
        mesh=plsc.VectorSubcoreMesh(core_axis_name="c", subcore_axis_name="s",
                                    num_cores=NC, num_subcores=NS),
        scratch_types=[
            pltpu.VMEM((spw * L,), jnp.int32),
            pltpu.VMEM((L, EMB // 2), jnp.int32),
            pltpu.VMEM((L, EMB // 2), jnp.int32),
            pltpu.VMEM((1, EMB), jnp.float32),
            pltpu.SemaphoreType.DMA,
            pltpu.SemaphoreType.DMA,
        ],
    )(_make_pool_body(spw))


def _mlp_body(x_ref, w1_ref, b1_ref, g_ref, bt_ref, w2_ref, b2_ref, o_ref):
    x = x_ref[...]
    h = jnp.dot(x, w1_ref[...], preferred_element_type=jnp.float32) + b1_ref[...]
    h = 0.5 * h * (1.0 + lax.erf(h * jnp.float32(0.7071067811865476)))
    mu = jnp.mean(h, axis=-1, keepdims=True)
    hc = h - mu
    var = jnp.mean(hc * hc, axis=-1, keepdims=True)
    h = hc * lax.rsqrt(var + 1e-5)
    h = h * g_ref[...] + bt_ref[...]
    out = jnp.dot(h, w2_ref[...], preferred_element_type=jnp.float32) + b2_ref[...]
    n2 = jnp.sum(out * out, axis=-1, keepdims=True)
    o_ref[...] = out * lax.rsqrt(jnp.maximum(n2, 1e-24))


def _pack_body(t_ref, o_ref):
    x = t_ref[...]
    bits = lax.bitcast_convert_type(x, jnp.uint32)
    rnd = (bits + jnp.uint32(0x7FFF) + ((bits >> 16) & jnp.uint32(1))) >> 16
    lo = rnd[:, :EMB // 2]
    hi_bits = bits[:, EMB // 2:]
    hi = (hi_bits + jnp.uint32(0x8000) - lo) >> 16
    o_ref[...] = lax.bitcast_convert_type(lo | (hi << 16), jnp.int32)


BLKV = 2000


def _pack(table):
    return pl.pallas_call(
        _pack_body,
        grid=(VOCAB // BLKV,),
        in_specs=[pl.BlockSpec((BLKV, EMB), lambda i: (i, 0))],
        out_specs=pl.BlockSpec((BLKV, EMB // 2), lambda i: (i, 0)),
        out_shape=jax.ShapeDtypeStruct((VOCAB, EMB // 2), jnp.int32),
    )(table)


BLK = 256


def _mlp_body_prev(prev_ref, x_ref, w1_ref, b1_ref, g_ref, bt_ref, w2_ref,
                   b2_ref, o_ref):
    del prev_ref
    _mlp_body(x_ref, w1_ref, b1_ref, g_ref, bt_ref, w2_ref, b2_ref, o_ref)


_W_SPECS = [
    pl.BlockSpec((EMB, HID), lambda i: (0, 0)),
    pl.BlockSpec((1, HID), lambda i: (0, 0)),
    pl.BlockSpec((1, HID), lambda i: (0, 0)),
    pl.BlockSpec((1, HID), lambda i: (0, 0)),
    pl.BlockSpec((HID, OUT), lambda i: (0, 0)),
    pl.BlockSpec((1, OUT), lambda i: (0, 0)),
]


def _mlp_chunk(prev, pooled_q, W1, b1, gamma, beta, W2, b2, q):
    nbq = pooled_q.shape[0]
    gq = nbq // BLK
    out_spec = pl.BlockSpec((BLK, OUT), lambda i, q=q, gq=gq: (i + q * gq, 0))
    x_spec = pl.BlockSpec((BLK, EMB), lambda i: (i, 0))
    if q == 0:
        return pl.pallas_call(
            _mlp_body,
            grid=(gq,),
            in_specs=[x_spec, *_W_SPECS],
            out_specs=out_spec,
            out_shape=jax.ShapeDtypeStruct((B, OUT), jnp.float32),
        )(pooled_q, W1, b1, gamma, beta, W2, b2)
    return pl.pallas_call(
        _mlp_body_prev,
        grid=(gq,),
        in_specs=[pl.BlockSpec(memory_space=pl.ANY), x_spec, *_W_SPECS],
        out_specs=out_spec,
        out_shape=jax.ShapeDtypeStruct((B, OUT), jnp.float32),
        input_output_aliases={0: 0},
    )(prev, pooled_q, W1, b1, gamma, beta, W2, b2)


NSPLIT = 4


def kernel(token_ids, table, W1, b1, gamma, beta, W2, b2):
    tok_flat = token_ids.reshape(-1).astype(jnp.int32)
    table_b = _pack(table)
    args = (W1, b1.reshape(1, HID), gamma.reshape(1, HID),
            beta.reshape(1, HID), W2, b2.reshape(1, OUT))
    nb = B // NSPLIT
    pooled = [_pool(nb)(tok_flat[i * nb * L:(i + 1) * nb * L], table_b)
              for i in range(NSPLIT)]
    out = None
    for q in range(NSPLIT):
        out = _mlp_chunk(out, pooled[q], *args, q)
    return out
